# baseline (speedup 1.0000x reference)
import functools

import jax
import jax.numpy as jnp
from jax import lax
from jax.experimental import pallas as pl
from jax.experimental.pallas import tpu as pltpu
from jax.experimental.pallas import tpu_sc as plsc

H = 2048
W = 2048
C = 16
B = 1048576

_NC = 2
_NS = 16
_NW = _NC * _NS
_L = 16

_TYW = (H // 8) // _NW
_AQ = _TYW * 16
_APX = 8 * 128


def _relayout_body(z_hbm, tbl_hbm, cbuf, obuf0, obuf1, sin0, sin1, sout0, sout1):
    wid = lax.axis_index("s") * _NC + lax.axis_index("c")
    ty_base = wid * _TYW
    lanes16 = lax.iota(jnp.int32, _L) * C

    def in_copies(q, par, sem):
        ty = ty_base + q // 16
        tx = q % 16
        return [pltpu.make_async_copy(
            z_hbm.at[c * (H // 8) + ty, tx],
            cbuf.at[pl.ds((par * C + c) * 1024, 1024)], sem) for c in range(C)]

    def out_copies(q, par, sem):
        ty = ty_base + q // 16
        tx = q % 16
        ob = obuf0 if par == 0 else obuf1
        return [pltpu.make_async_copy(
            ob.at[pl.ds(r * (128 * C), 128 * C)],
            tbl_hbm.at[pl.ds(((ty * 8 + r) * W + tx * 128) * C, 128 * C)],
            sem) for r in range(8)]

    iotaL = lax.iota(jnp.int32, _L)
    low3 = iotaL & 7
    hi1 = iotaL >> 3
    _AK = [((iotaL + k) & 15) * 8 + low3 for k in range(_L)]
    _GK = [a + hi1 * 8192 for a in _AK]
    _SK = [a * C + hi1 * 8 for a in _AK]

    def interleave(par):
        dst = obuf0 if par == 0 else obuf1
        par_off = par * (C * 1024)

        @plsc.parallel_loop(0, 64, unroll=2)
        def _(i):
            r = i >> 3
            c0 = i & 7
            g0 = par_off + c0 * 1024 + r * 128
            s0 = r * 2048 + c0
            for k in range(_L):
                vals = plsc.load_gather(cbuf, [_GK[k] + g0])
                plsc.store_scatter(dst, [_SK[k] + s0], vals)

    for cp in in_copies(0, 0, sin0):
        cp.start()

    def qq_body(qq, _):
        for par, sin, sout in ((0, sin0, sout0), (1, sin1, sout1)):
            q = qq * 2 + par
            nsin = sin1 if par == 0 else sin0
            @pl.when(q + 1 < _AQ)
            def _():
                for cp in in_copies(q + 1, 1 - par, nsin):
                    cp.start()
            for cp in in_copies(q, par, sin):
                cp.wait()
            @pl.when(qq >= 1)
            def _():
                for cp in out_copies(q - 2, par, sout):
                    cp.wait()
            interleave(par)
            for cp in out_copies(q, par, sout):
                cp.start()
        return 0

    lax.fori_loop(0, _AQ // 2, qq_body, 0)
    for cp in out_copies(_AQ - 2, 0, sout0):
        cp.wait()
    for cp in out_copies(_AQ - 1, 1, sout1):
        cp.wait()


_relayout_call = functools.partial(
    pl.kernel,
    mesh=plsc.VectorSubcoreMesh(core_axis_name="c", subcore_axis_name="s"),
    out_type=jax.ShapeDtypeStruct((H * W * C,), jnp.float32),
    compiler_params=pltpu.CompilerParams(use_tc_tiling_on_sc=False, needs_layout_passes=False),
    scratch_types=[
        pltpu.VMEM((2 * C * 1024,), jnp.float32),
        pltpu.VMEM((_APX * C,), jnp.float32),
        pltpu.VMEM((_APX * C,), jnp.float32),
        pltpu.SemaphoreType.DMA,
        pltpu.SemaphoreType.DMA,
        pltpu.SemaphoreType.DMA,
        pltpu.SemaphoreType.DMA,
    ],
)(_relayout_body)

_BPW = B // _NW
_CH = 512
_G = _BPW // _CH


def _gather_body(tbl_hbm, u_hbm, v_hbm, out_hbm, *scr):
    (ub0, vb0, ub1, vb1) = scr[0:4]
    ib = (scr[4:8], scr[8:12])
    wb = (scr[12:16], scr[16:20])
    tp = (scr[20:24], scr[24:28])
    ob = scr[28:30]
    suv = scr[30:32]
    sg = scr[32:34]
    so = scr[34:36]
    uvb = ((ub0, vb0), (ub1, vb1))
    wid = lax.axis_index("s") * _NC + lax.axis_index("c")
    base = wid * _BPW
    iota16 = lax.iota(jnp.int32, _L)

    dnums = lax.GatherDimensionNumbers(
        offset_dims=(), collapsed_slice_dims=(0,), start_index_map=(0,))

    def _splat(vec, j):
        idxs = jnp.full((_L, 1), j, dtype=jnp.int32)
        return lax.gather(vec, idxs, dnums, slice_sizes=(1,),
                          mode=lax.GatherScatterMode.PROMISE_IN_BOUNDS)

    def uv_copies(g, par):
        off = base + g * _CH
        return [pltpu.make_async_copy(u_hbm.at[pl.ds(off, _CH)],
                                      uvb[par][0], suv[par]),
                pltpu.make_async_copy(v_hbm.at[pl.ds(off, _CH)],
                                      uvb[par][1], suv[par])]

    def gather_copies(par):
        return [pltpu.make_async_copy(tbl_hbm.at[ib[par][t]], tp[par][t],
                                      sg[par]) for t in range(4)]

    def out_copies(g, par):
        pb0 = (base + g * _CH) // 128
        return [pltpu.make_async_copy(
            ob[par].at[cb * 8 + cl, :, pl.ds(0, 128)],
            out_hbm.at[cb, pl.ds(pb0, _CH // 128), cl, :],
            so[par]) for cb in range(2) for cl in range(8)]

    def build(g, par):
        ubuf, vbuf = uvb[par]
        ib0, ib1, ib2, ib3 = ib[par]
        wb0, wb1, wb2, wb3 = wb[par]

        def grp(i):
            s = i * _L
            u = ubuf[pl.ds(s, _L)]
            v = vbuf[pl.ds(s, _L)]
            gx = u * 2.0 - 1.0
            gy = v * 2.0 - 1.0
            ix = ((gx + 1.0) * W - 1.0) * 0.5
            iy = ((gy + 1.0) * H - 1.0) * 0.5
            ix = jnp.minimum(jnp.maximum(ix, 0.0), W - 1.0)
            iy = jnp.minimum(jnp.maximum(iy, 0.0), H - 1.0)
            x0 = ix.astype(jnp.int32)
            y0 = iy.astype(jnp.int32)
            wx1 = ix - x0.astype(jnp.float32)
            wy1 = iy - y0.astype(jnp.float32)
            wx0 = 1.0 - wx1
            wy0 = 1.0 - wy1
            x1 = jnp.minimum(x0 + 1, W - 1)
            y1 = jnp.minimum(y0 + 1, H - 1)
            r0 = y0 * W
            r1 = y1 * W
            ib0[pl.ds(s, _L)] = r0 + x0
            ib1[pl.ds(s, _L)] = r0 + x1
            ib2[pl.ds(s, _L)] = r1 + x0
            ib3[pl.ds(s, _L)] = r1 + x1
            wb0[pl.ds(s, _L)] = wy0 * wx0
            wb1[pl.ds(s, _L)] = wy0 * wx1
            wb2[pl.ds(s, _L)] = wy1 * wx0
            wb3[pl.ds(s, _L)] = wy1 * wx1

        plsc.parallel_loop(0, _CH // _L, unroll=4)(grp)

    def combine(par):
        tp0, tp1, tp2, tp3 = tp[par]
        wb0, wb1, wb2, wb3 = wb[par]
        obuf3 = ob[par]

        def grp2(i):
            s = i * _L
            w0 = wb0[pl.ds(s, _L)]
            w1 = wb1[pl.ds(s, _L)]
            w2 = wb2[pl.ds(s, _L)]
            w3 = wb3[pl.ds(s, _L)]
            rowv = iota16 + s
            pbv = jnp.full((_L,), i // 8, jnp.int32)
            plv = iota16 + (s % 128)
            for c in range(C):
                diag = (iota16 + c) & (C - 1)
                t0 = plsc.load_gather(tp0, [rowv, diag])
                t1 = plsc.load_gather(tp1, [rowv, diag])
                t2 = plsc.load_gather(tp2, [rowv, diag])
                t3 = plsc.load_gather(tp3, [rowv, diag])
                ocol = t0 * w0 + t1 * w1 + t2 * w2 + t3 * w3
                plsc.store_scatter(obuf3, [diag, pbv, plv], ocol)

        plsc.parallel_loop(0, _CH // _L, unroll=2)(grp2)

    for cp in uv_copies(0, 0):
        cp.start()

    def qq_body(qq, _):
        for par in (0, 1):
            g = qq * 2 + par
            for cp in uv_copies(g, par):
                cp.wait()
            build(g, par)
            for cp in gather_copies(par):
                cp.start()
            @pl.when(g + 1 < _G)
            def _():
                for cp in uv_copies(g + 1, 1 - par):
                    cp.start()
            @pl.when(g >= 1)
            def _():
                for cp in gather_copies(1 - par):
                    cp.wait()
                @pl.when(g >= 3)
                def _():
                    for cp in out_copies(g - 3, 1 - par):
                        cp.wait()
                combine(1 - par)
                for cp in out_copies(g - 1, 1 - par):
                    cp.start()
        return 0

    lax.fori_loop(0, _G // 2, qq_body, 0)
    for cp in gather_copies(1):
        cp.wait()
    for cp in out_copies(_G - 3, 1):
        cp.wait()
    combine(1)
    for cp in out_copies(_G - 1, 1):
        cp.start()
    for cp in out_copies(_G - 2, 0):
        cp.wait()
    for cp in out_copies(_G - 1, 1):
        cp.wait()


_gather_call = functools.partial(
    pl.kernel,
    mesh=plsc.VectorSubcoreMesh(core_axis_name="c", subcore_axis_name="s"),
    out_type=jax.ShapeDtypeStruct((2, B // 128, 8, 128), jnp.float32),
    compiler_params=pltpu.CompilerParams(use_tc_tiling_on_sc=False, needs_layout_passes=False),
    scratch_types=(
        [pltpu.VMEM((_CH,), jnp.float32)] * 4
        + [pltpu.VMEM((_CH,), jnp.int32)] * 8
        + [pltpu.VMEM((_CH,), jnp.float32)] * 8
        + [pltpu.VMEM((_CH, C), jnp.float32)] * 8
        + [pltpu.VMEM((C, _CH // 128, 130), jnp.float32)] * 2
        + [pltpu.SemaphoreType.DMA] * 6
    ),
)(_gather_body)


def kernel(uv, Z):
    u = uv[:, 0]
    v = uv[:, 1]
    zt = (Z[0].reshape(16, 256, 8, 16, 128).transpose(0, 1, 3, 2, 4)
          .reshape(4096, 16, 1024))
    tbl = _relayout_call(zt)
    out4 = _gather_call(tbl.reshape(H * W, C), u, v)
    return out4.transpose(1, 3, 0, 2).reshape(B, C)


# speedup vs baseline: 1.2580x; 1.2580x over previous
import functools

import jax
import jax.numpy as jnp
from jax import lax
from jax.experimental import pallas as pl
from jax.experimental.pallas import tpu as pltpu
from jax.experimental.pallas import tpu_sc as plsc

H = 2048
W = 2048
C = 16
B = 1048576

_NC = 2
_NS = 16
_NW = _NC * _NS
_L = 16

_TYW = (H // 8) // _NW
_AQ = _TYW * 16
_APX = 8 * 128


def _relayout_body(z_hbm, tbl_hbm, cbuf, obuf0, obuf1, sin0, sin1, sout0, sout1):
    wid = lax.axis_index("s") * _NC + lax.axis_index("c")
    ty_base = wid * _TYW
    lanes16 = lax.iota(jnp.int32, _L) * C

    def in_copies(q, par, sem):
        ty = ty_base + q // 16
        tx = q % 16
        return [pltpu.make_async_copy(
            z_hbm.at[c * (H // 8) + ty, tx],
            cbuf.at[pl.ds((par * C + c) * 1024, 1024)], sem) for c in range(C)]

    def out_copies(q, par, sem):
        ty = ty_base + q // 16
        tx = q % 16
        ob = obuf0 if par == 0 else obuf1
        return [pltpu.make_async_copy(
            ob.at[pl.ds(r * (128 * C), 128 * C)],
            tbl_hbm.at[pl.ds(((ty * 8 + r) * W + tx * 128) * C, 128 * C)],
            sem) for r in range(8)]

    iotaL = lax.iota(jnp.int32, _L)
    low3 = iotaL & 7
    hi1 = iotaL >> 3
    _AK = [((iotaL + k) & 15) * 8 + low3 for k in range(_L)]
    _GK = [a + hi1 * 8192 for a in _AK]
    _SK = [a * C + hi1 * 8 for a in _AK]

    def interleave(par):
        dst = obuf0 if par == 0 else obuf1
        par_off = par * (C * 1024)

        @plsc.parallel_loop(0, 64, unroll=2)
        def _(i):
            r = i >> 3
            c0 = i & 7
            g0 = par_off + c0 * 1024 + r * 128
            s0 = r * 2048 + c0
            for k in range(_L):
                vals = plsc.load_gather(cbuf, [_GK[k] + g0])
                plsc.store_scatter(dst, [_SK[k] + s0], vals)

    for cp in in_copies(0, 0, sin0):
        cp.start()

    def qq_body(qq, _):
        for par, sin, sout in ((0, sin0, sout0), (1, sin1, sout1)):
            q = qq * 2 + par
            nsin = sin1 if par == 0 else sin0
            @pl.when(q + 1 < _AQ)
            def _():
                for cp in in_copies(q + 1, 1 - par, nsin):
                    cp.start()
            for cp in in_copies(q, par, sin):
                cp.wait()
            @pl.when(qq >= 1)
            def _():
                for cp in out_copies(q - 2, par, sout):
                    cp.wait()
            interleave(par)
            for cp in out_copies(q, par, sout):
                cp.start()
        return 0

    lax.fori_loop(0, _AQ // 2, qq_body, 0)
    for cp in out_copies(_AQ - 2, 0, sout0):
        cp.wait()
    for cp in out_copies(_AQ - 1, 1, sout1):
        cp.wait()


_relayout_call = functools.partial(
    pl.kernel,
    mesh=plsc.VectorSubcoreMesh(core_axis_name="c", subcore_axis_name="s"),
    out_type=jax.ShapeDtypeStruct((H * W * C,), jnp.float32),
    compiler_params=pltpu.CompilerParams(use_tc_tiling_on_sc=False, needs_layout_passes=False),
    scratch_types=[
        pltpu.VMEM((2 * C * 1024,), jnp.float32),
        pltpu.VMEM((_APX * C,), jnp.float32),
        pltpu.VMEM((_APX * C,), jnp.float32),
        pltpu.SemaphoreType.DMA,
        pltpu.SemaphoreType.DMA,
        pltpu.SemaphoreType.DMA,
        pltpu.SemaphoreType.DMA,
    ],
)(_relayout_body)

_BPW = B // _NW
_CH = 512
_G = _BPW // _CH


def _gather_body(tbl_hbm, u_hbm, v_hbm, out_hbm, *scr):
    (ub0, vb0, ub1, vb1) = scr[0:4]
    ib = (scr[4:8], scr[8:12])
    wb = (scr[12:16], scr[16:20])
    tp = (scr[20:24], scr[24:28])
    ob = scr[28:30]
    suv = scr[30:32]
    sg = scr[32:34]
    so = scr[34:36]
    uvb = ((ub0, vb0), (ub1, vb1))
    wid = lax.axis_index("s") * _NC + lax.axis_index("c")
    base = wid * _BPW
    iota16 = lax.iota(jnp.int32, _L)

    dnums = lax.GatherDimensionNumbers(
        offset_dims=(), collapsed_slice_dims=(0,), start_index_map=(0,))

    def _splat(vec, j):
        idxs = jnp.full((_L, 1), j, dtype=jnp.int32)
        return lax.gather(vec, idxs, dnums, slice_sizes=(1,),
                          mode=lax.GatherScatterMode.PROMISE_IN_BOUNDS)

    def uv_copies(g, par):
        off = base + g * _CH
        return [pltpu.make_async_copy(u_hbm.at[pl.ds(off, _CH)],
                                      uvb[par][0], suv[par]),
                pltpu.make_async_copy(v_hbm.at[pl.ds(off, _CH)],
                                      uvb[par][1], suv[par])]

    def gather_copies(par):
        return [pltpu.make_async_copy(tbl_hbm.at[ib[par][t]], tp[par][t],
                                      sg[par]) for t in range(4)]

    def out_copies(g, par):
        pb0 = (base + g * _CH) // 128
        return [pltpu.make_async_copy(
            ob[par].at[cb * 8 + cl, :, pl.ds(0, 128)],
            out_hbm.at[cb, pl.ds(pb0, _CH // 128), cl, :],
            so[par]) for cb in range(2) for cl in range(8)]

    def build(g, par):
        ubuf, vbuf = uvb[par]
        ib0, ib1, ib2, ib3 = ib[par]
        wb0, wb1, wb2, wb3 = wb[par]

        def grp(i):
            s = i * _L
            u = ubuf[pl.ds(s, _L)]
            v = vbuf[pl.ds(s, _L)]
            gx = u * 2.0 - 1.0
            gy = v * 2.0 - 1.0
            ix = ((gx + 1.0) * W - 1.0) * 0.5
            iy = ((gy + 1.0) * H - 1.0) * 0.5
            ix = jnp.minimum(jnp.maximum(ix, 0.0), W - 1.0)
            iy = jnp.minimum(jnp.maximum(iy, 0.0), H - 1.0)
            x0 = ix.astype(jnp.int32)
            y0 = iy.astype(jnp.int32)
            wx1 = ix - x0.astype(jnp.float32)
            wy1 = iy - y0.astype(jnp.float32)
            wx0 = 1.0 - wx1
            wy0 = 1.0 - wy1
            x1 = jnp.minimum(x0 + 1, W - 1)
            y1 = jnp.minimum(y0 + 1, H - 1)
            r0 = y0 * W
            r1 = y1 * W
            ib0[pl.ds(s, _L)] = r0 + x0
            ib1[pl.ds(s, _L)] = r0 + x1
            ib2[pl.ds(s, _L)] = r1 + x0
            ib3[pl.ds(s, _L)] = r1 + x1
            wb0[pl.ds(s, _L)] = wy0 * wx0
            wb1[pl.ds(s, _L)] = wy0 * wx1
            wb2[pl.ds(s, _L)] = wy1 * wx0
            wb3[pl.ds(s, _L)] = wy1 * wx1

        plsc.parallel_loop(0, _CH // _L, unroll=2)(grp)

    def combine(par):
        tp0, tp1, tp2, tp3 = tp[par]
        wb0, wb1, wb2, wb3 = wb[par]
        obuf3 = ob[par]

        def grp2(i):
            s = i * _L
            w0 = wb0[pl.ds(s, _L)]
            w1 = wb1[pl.ds(s, _L)]
            w2 = wb2[pl.ds(s, _L)]
            w3 = wb3[pl.ds(s, _L)]
            rowv = iota16 + s
            pbv = jnp.full((_L,), i // 8, jnp.int32)
            plv = iota16 + (s % 128)
            for c in range(C):
                diag = (iota16 + c) & (C - 1)
                t0 = plsc.load_gather(tp0, [rowv, diag])
                t1 = plsc.load_gather(tp1, [rowv, diag])
                t2 = plsc.load_gather(tp2, [rowv, diag])
                t3 = plsc.load_gather(tp3, [rowv, diag])
                ocol = t0 * w0 + t1 * w1 + t2 * w2 + t3 * w3
                plsc.store_scatter(obuf3, [diag, pbv, plv], ocol)

        plsc.parallel_loop(0, _CH // _L, unroll=1)(grp2)

    for cp in uv_copies(0, 0):
        cp.start()

    def qq_body(qq, _):
        for par in (0, 1):
            g = qq * 2 + par
            for cp in uv_copies(g, par):
                cp.wait()
            build(g, par)
            for cp in gather_copies(par):
                cp.start()
            @pl.when(g + 1 < _G)
            def _():
                for cp in uv_copies(g + 1, 1 - par):
                    cp.start()
            @pl.when(g >= 1)
            def _():
                for cp in gather_copies(1 - par):
                    cp.wait()
                @pl.when(g >= 3)
                def _():
                    for cp in out_copies(g - 3, 1 - par):
                        cp.wait()
                combine(1 - par)
                for cp in out_copies(g - 1, 1 - par):
                    cp.start()
        return 0

    lax.fori_loop(0, _G // 2, qq_body, 0)
    for cp in gather_copies(1):
        cp.wait()
    for cp in out_copies(_G - 3, 1):
        cp.wait()
    combine(1)
    for cp in out_copies(_G - 1, 1):
        cp.start()
    for cp in out_copies(_G - 2, 0):
        cp.wait()
    for cp in out_copies(_G - 1, 1):
        cp.wait()


_gather_call = functools.partial(
    pl.kernel,
    mesh=plsc.VectorSubcoreMesh(core_axis_name="c", subcore_axis_name="s"),
    out_type=jax.ShapeDtypeStruct((2, B // 128, 8, 128), jnp.float32),
    compiler_params=pltpu.CompilerParams(use_tc_tiling_on_sc=False, needs_layout_passes=False),
    scratch_types=(
        [pltpu.VMEM((_CH,), jnp.float32)] * 4
        + [pltpu.VMEM((_CH,), jnp.int32)] * 8
        + [pltpu.VMEM((_CH,), jnp.float32)] * 8
        + [pltpu.VMEM((_CH, C), jnp.float32)] * 8
        + [pltpu.VMEM((C, _CH // 128, 130), jnp.float32)] * 2
        + [pltpu.SemaphoreType.DMA] * 6
    ),
)(_gather_body)


def kernel(uv, Z):
    u = uv[:, 0]
    v = uv[:, 1]
    zt = (Z[0].reshape(16, 256, 8, 16, 128).transpose(0, 1, 3, 2, 4)
          .reshape(4096, 16, 1024))
    tbl = _relayout_call(zt)
    out4 = _gather_call(tbl.reshape(H * W, C), u, v)
    return out4.transpose(1, 3, 0, 2).reshape(B, C)


# speedup vs baseline: 1.2701x; 1.0096x over previous
import functools

import jax
import jax.numpy as jnp
from jax import lax
from jax.experimental import pallas as pl
from jax.experimental.pallas import tpu as pltpu
from jax.experimental.pallas import tpu_sc as plsc

H = 2048
W = 2048
C = 16
B = 1048576

_NC = 2
_NS = 16
_NW = _NC * _NS
_L = 16

_TYW = (H // 8) // _NW
_AQ = _TYW * 16
_APX = 8 * 128


def _relayout_body(z_hbm, tbl_hbm, cbuf, obuf0, obuf1, sin0, sin1, sout0, sout1):
    wid = lax.axis_index("s") * _NC + lax.axis_index("c")
    ty_base = wid * _TYW
    lanes16 = lax.iota(jnp.int32, _L) * C

    def in_copies(q, par, sem):
        ty = ty_base + q // 16
        tx = q % 16
        return [pltpu.make_async_copy(
            z_hbm.at[c * (H // 8) + ty, tx],
            cbuf.at[pl.ds((par * C + c) * 1024, 1024)], sem) for c in range(C)]

    def out_copies(q, par, sem):
        ty = ty_base + q // 16
        tx = q % 16
        ob = obuf0 if par == 0 else obuf1
        return [pltpu.make_async_copy(
            ob.at[pl.ds(r * (128 * C), 128 * C)],
            tbl_hbm.at[pl.ds(((ty * 8 + r) * W + tx * 128) * C, 128 * C)],
            sem) for r in range(8)]

    iotaL = lax.iota(jnp.int32, _L)
    low3 = iotaL & 7
    hi1 = iotaL >> 3
    _AK = [((iotaL + k) & 15) * 8 + low3 for k in range(_L)]
    _GK = [a + hi1 * 8192 for a in _AK]
    _SK = [a * C + hi1 * 8 for a in _AK]

    def interleave(par):
        dst = obuf0 if par == 0 else obuf1
        par_off = par * (C * 1024)

        @plsc.parallel_loop(0, 64, unroll=4)
        def _(i):
            r = i >> 3
            c0 = i & 7
            g0 = par_off + c0 * 1024 + r * 128
            s0 = r * 2048 + c0
            for k in range(_L):
                vals = plsc.load_gather(cbuf, [_GK[k] + g0])
                plsc.store_scatter(dst, [_SK[k] + s0], vals)

    for cp in in_copies(0, 0, sin0):
        cp.start()

    def qq_body(qq, _):
        for par, sin, sout in ((0, sin0, sout0), (1, sin1, sout1)):
            q = qq * 2 + par
            nsin = sin1 if par == 0 else sin0
            @pl.when(q + 1 < _AQ)
            def _():
                for cp in in_copies(q + 1, 1 - par, nsin):
                    cp.start()
            for cp in in_copies(q, par, sin):
                cp.wait()
            @pl.when(qq >= 1)
            def _():
                for cp in out_copies(q - 2, par, sout):
                    cp.wait()
            interleave(par)
            for cp in out_copies(q, par, sout):
                cp.start()
        return 0

    lax.fori_loop(0, _AQ // 2, qq_body, 0)
    for cp in out_copies(_AQ - 2, 0, sout0):
        cp.wait()
    for cp in out_copies(_AQ - 1, 1, sout1):
        cp.wait()


_relayout_call = functools.partial(
    pl.kernel,
    mesh=plsc.VectorSubcoreMesh(core_axis_name="c", subcore_axis_name="s"),
    out_type=jax.ShapeDtypeStruct((H * W * C,), jnp.float32),
    compiler_params=pltpu.CompilerParams(use_tc_tiling_on_sc=False, needs_layout_passes=False),
    scratch_types=[
        pltpu.VMEM((2 * C * 1024,), jnp.float32),
        pltpu.VMEM((_APX * C,), jnp.float32),
        pltpu.VMEM((_APX * C,), jnp.float32),
        pltpu.SemaphoreType.DMA,
        pltpu.SemaphoreType.DMA,
        pltpu.SemaphoreType.DMA,
        pltpu.SemaphoreType.DMA,
    ],
)(_relayout_body)

_BPW = B // _NW
_CH = 512
_G = _BPW // _CH


def _gather_body(tbl_hbm, u_hbm, v_hbm, out_hbm, *scr):
    (ub0, vb0, ub1, vb1) = scr[0:4]
    ib = (scr[4:8], scr[8:12])
    wb = (scr[12:16], scr[16:20])
    tp = (scr[20:24], scr[24:28])
    ob = scr[28:30]
    suv = scr[30:32]
    sg = scr[32:34]
    so = scr[34:36]
    uvb = ((ub0, vb0), (ub1, vb1))
    wid = lax.axis_index("s") * _NC + lax.axis_index("c")
    base = wid * _BPW
    iota16 = lax.iota(jnp.int32, _L)

    dnums = lax.GatherDimensionNumbers(
        offset_dims=(), collapsed_slice_dims=(0,), start_index_map=(0,))

    def _splat(vec, j):
        idxs = jnp.full((_L, 1), j, dtype=jnp.int32)
        return lax.gather(vec, idxs, dnums, slice_sizes=(1,),
                          mode=lax.GatherScatterMode.PROMISE_IN_BOUNDS)

    def uv_copies(g, par):
        off = base + g * _CH
        return [pltpu.make_async_copy(u_hbm.at[pl.ds(off, _CH)],
                                      uvb[par][0], suv[par]),
                pltpu.make_async_copy(v_hbm.at[pl.ds(off, _CH)],
                                      uvb[par][1], suv[par])]

    def gather_copies(par):
        return [pltpu.make_async_copy(tbl_hbm.at[ib[par][t]], tp[par][t],
                                      sg[par]) for t in range(4)]

    def out_copies(g, par):
        pb0 = (base + g * _CH) // 128
        return [pltpu.make_async_copy(
            ob[par].at[cb * 8 + cl, :, pl.ds(0, 128)],
            out_hbm.at[cb, pl.ds(pb0, _CH // 128), cl, :],
            so[par]) for cb in range(2) for cl in range(8)]

    def build(g, par):
        ubuf, vbuf = uvb[par]
        ib0, ib1, ib2, ib3 = ib[par]
        wb0, wb1, wb2, wb3 = wb[par]

        def grp(i):
            s = i * _L
            u = ubuf[pl.ds(s, _L)]
            v = vbuf[pl.ds(s, _L)]
            gx = u * 2.0 - 1.0
            gy = v * 2.0 - 1.0
            ix = ((gx + 1.0) * W - 1.0) * 0.5
            iy = ((gy + 1.0) * H - 1.0) * 0.5
            ix = jnp.minimum(jnp.maximum(ix, 0.0), W - 1.0)
            iy = jnp.minimum(jnp.maximum(iy, 0.0), H - 1.0)
            x0 = ix.astype(jnp.int32)
            y0 = iy.astype(jnp.int32)
            wx1 = ix - x0.astype(jnp.float32)
            wy1 = iy - y0.astype(jnp.float32)
            wx0 = 1.0 - wx1
            wy0 = 1.0 - wy1
            x1 = jnp.minimum(x0 + 1, W - 1)
            y1 = jnp.minimum(y0 + 1, H - 1)
            r0 = y0 * W
            r1 = y1 * W
            ib0[pl.ds(s, _L)] = r0 + x0
            ib1[pl.ds(s, _L)] = r0 + x1
            ib2[pl.ds(s, _L)] = r1 + x0
            ib3[pl.ds(s, _L)] = r1 + x1
            wb0[pl.ds(s, _L)] = wy0 * wx0
            wb1[pl.ds(s, _L)] = wy0 * wx1
            wb2[pl.ds(s, _L)] = wy1 * wx0
            wb3[pl.ds(s, _L)] = wy1 * wx1

        plsc.parallel_loop(0, _CH // _L, unroll=2)(grp)

    def combine(par):
        tp0, tp1, tp2, tp3 = tp[par]
        wb0, wb1, wb2, wb3 = wb[par]
        obuf3 = ob[par]

        def grp2(i):
            s = i * _L
            w0 = wb0[pl.ds(s, _L)]
            w1 = wb1[pl.ds(s, _L)]
            w2 = wb2[pl.ds(s, _L)]
            w3 = wb3[pl.ds(s, _L)]
            rowv = iota16 + s
            pbv = jnp.full((_L,), i // 8, jnp.int32)
            plv = iota16 + (s % 128)
            for c in range(C):
                diag = (iota16 + c) & (C - 1)
                t0 = plsc.load_gather(tp0, [rowv, diag])
                t1 = plsc.load_gather(tp1, [rowv, diag])
                t2 = plsc.load_gather(tp2, [rowv, diag])
                t3 = plsc.load_gather(tp3, [rowv, diag])
                ocol = t0 * w0 + t1 * w1 + t2 * w2 + t3 * w3
                plsc.store_scatter(obuf3, [diag, pbv, plv], ocol)

        plsc.parallel_loop(0, _CH // _L, unroll=1)(grp2)

    for cp in uv_copies(0, 0):
        cp.start()

    def qq_body(qq, _):
        for par in (0, 1):
            g = qq * 2 + par
            for cp in uv_copies(g, par):
                cp.wait()
            build(g, par)
            for cp in gather_copies(par):
                cp.start()
            @pl.when(g + 1 < _G)
            def _():
                for cp in uv_copies(g + 1, 1 - par):
                    cp.start()
            @pl.when(g >= 1)
            def _():
                for cp in gather_copies(1 - par):
                    cp.wait()
                @pl.when(g >= 3)
                def _():
                    for cp in out_copies(g - 3, 1 - par):
                        cp.wait()
                combine(1 - par)
                for cp in out_copies(g - 1, 1 - par):
                    cp.start()
        return 0

    lax.fori_loop(0, _G // 2, qq_body, 0)
    for cp in gather_copies(1):
        cp.wait()
    for cp in out_copies(_G - 3, 1):
        cp.wait()
    combine(1)
    for cp in out_copies(_G - 1, 1):
        cp.start()
    for cp in out_copies(_G - 2, 0):
        cp.wait()
    for cp in out_copies(_G - 1, 1):
        cp.wait()


_gather_call = functools.partial(
    pl.kernel,
    mesh=plsc.VectorSubcoreMesh(core_axis_name="c", subcore_axis_name="s"),
    out_type=jax.ShapeDtypeStruct((2, B // 128, 8, 128), jnp.float32),
    compiler_params=pltpu.CompilerParams(use_tc_tiling_on_sc=False, needs_layout_passes=False),
    scratch_types=(
        [pltpu.VMEM((_CH,), jnp.float32)] * 4
        + [pltpu.VMEM((_CH,), jnp.int32)] * 8
        + [pltpu.VMEM((_CH,), jnp.float32)] * 8
        + [pltpu.VMEM((_CH, C), jnp.float32)] * 8
        + [pltpu.VMEM((C, _CH // 128, 130), jnp.float32)] * 2
        + [pltpu.SemaphoreType.DMA] * 6
    ),
)(_gather_body)


def kernel(uv, Z):
    u = uv[:, 0]
    v = uv[:, 1]
    zt = (Z[0].reshape(16, 256, 8, 16, 128).transpose(0, 1, 3, 2, 4)
          .reshape(4096, 16, 1024))
    tbl = _relayout_call(zt)
    out4 = _gather_call(tbl.reshape(H * W, C), u, v)
    return out4.transpose(1, 3, 0, 2).reshape(B, C)
